# Initial kernel scaffold; baseline (speedup 1.0000x reference)
import functools

import jax
import jax.numpy as jnp
from jax import lax
from jax.experimental import pallas as pl
from jax.experimental.pallas import tpu as pltpu
from jax.experimental.pallas import tpu_sc as plsc

_LANES = 16


def _sc_gather_partials(x2d, tgt, B, C):
    NC, NS = 2, 16
    NW = NC * NS
    BPW = B // NW
    NCHUNK = BPW // _LANES
    NGATHER = BPW // 128

    mesh = plsc.VectorSubcoreMesh(core_axis_name="c", subcore_axis_name="s")

    @functools.partial(
        pl.kernel,
        out_type=jax.ShapeDtypeStruct((NW, _LANES), jnp.float32),
        mesh=mesh,
        scratch_types=[
            pltpu.VMEM((BPW,), jnp.int32),
            pltpu.VMEM((NGATHER, 128), jnp.int32),
            pltpu.VMEM((BPW,), jnp.int32),
            pltpu.VMEM((BPW, _LANES), jnp.float32),
            pltpu.VMEM((_LANES,), jnp.float32),
            pltpu.SemaphoreType.DMA,
        ],
    )
    def sc_kernel(x_hbm, tgt_hbm, out_hbm, tgt_v, idx_v, lane_v, rows_v,
                  acc_v, sem):
        wid = lax.axis_index("s") * NC + lax.axis_index("c")
        base = wid * BPW

        pltpu.sync_copy(tgt_hbm.at[pl.ds(base, BPW)], tgt_v)

        lane_iota = lax.iota(jnp.int32, _LANES)
        for j in range(NCHUNK):
            t = tgt_v[pl.ds(j * _LANES, _LANES)]
            rows = base + j * _LANES + lane_iota
            flat = rows * C + t
            idx_v[j // 8, pl.ds((j % 8) * _LANES, _LANES)] = (
                lax.shift_right_logical(flat, 4))
            lane_v[pl.ds(j * _LANES, _LANES)] = lax.bitwise_and(flat, 15)

        copies = [
            pltpu.async_copy(
                x_hbm.at[idx_v.at[k]],
                rows_v.at[pl.ds(k * 128, 128)],
                sem,
            )
            for k in range(NGATHER)
        ]
        for cp in copies:
            cp.wait()

        acc = jnp.zeros((_LANES,), jnp.float32)
        for j in range(NCHUNK):
            row_loc = j * _LANES + lane_iota
            lanes = lane_v[pl.ds(j * _LANES, _LANES)]
            acc = acc + plsc.load_gather(rows_v, [row_loc, lanes])
        acc_v[...] = acc
        pltpu.sync_copy(acc_v, out_hbm.at[wid])

    return sc_kernel(x2d, tgt)


def _tc_loss(x, partials, B, C):
    GRID = 32
    BR = B // GRID
    inv_n = 1.0 / (B * C)

    def body(x_ref, p_ref, o_ref):
        i = pl.program_id(0)

        @pl.when(i == 0)
        def _init():
            o_ref[0, 0] = 0.0

        xb = x_ref[...]
        o_ref[0, 0] += jnp.sum(xb * xb)

        @pl.when(i == GRID - 1)
        def _finish():
            gsum = jnp.sum(p_ref[...])
            o_ref[0, 0] = (o_ref[0, 0] - 2.0 * gsum + float(B)) * inv_n

    return pl.pallas_call(
        body,
        grid=(GRID,),
        in_specs=[
            pl.BlockSpec((BR, C), lambda i: (i, 0)),
            pl.BlockSpec(partials.shape, lambda i: (0, 0)),
        ],
        out_specs=pl.BlockSpec(memory_space=pltpu.SMEM),
        out_shape=jax.ShapeDtypeStruct((1, 1), jnp.float32),
    )(x, partials)


def kernel(input_score, target):
    B, C = input_score.shape
    tgt = target.reshape(-1).astype(jnp.int32)
    x2d = input_score.reshape(B * C // _LANES, _LANES)
    partials = _sc_gather_partials(x2d, tgt, B, C)
    loss = _tc_loss(input_score, partials, B, C)
    return loss[0, 0]


# speedup vs baseline: 1.1218x; 1.1218x over previous
import functools

import jax
import jax.numpy as jnp
from jax import lax
from jax.experimental import pallas as pl
from jax.experimental.pallas import tpu as pltpu
from jax.experimental.pallas import tpu_sc as plsc

_LANES = 16


def _sc_gather_partials(x_flat, tgt, B, C):
    NC, NS = 2, 16
    NW = NC * NS
    BPW = B // NW
    NCHUNK = BPW // _LANES
    NGATHER = BPW // 128

    mesh = plsc.VectorSubcoreMesh(core_axis_name="c", subcore_axis_name="s")

    @functools.partial(
        pl.kernel,
        out_type=jax.ShapeDtypeStruct((NW, _LANES), jnp.float32),
        mesh=mesh,
        scratch_types=[
            pltpu.VMEM((BPW,), jnp.int32),
            pltpu.VMEM((NGATHER, 128), jnp.int32),
            pltpu.VMEM((BPW,), jnp.float32),
            pltpu.VMEM((_LANES,), jnp.float32),
            pltpu.SemaphoreType.DMA,
        ],
    )
    def sc_kernel(x_hbm, tgt_hbm, out_hbm, tgt_v, idx_v, got_v, acc_v, sem):
        wid = lax.axis_index("s") * NC + lax.axis_index("c")
        base = wid * BPW

        pltpu.sync_copy(tgt_hbm.at[pl.ds(base, BPW)], tgt_v)

        lane_iota = lax.iota(jnp.int32, _LANES)
        for j in range(NCHUNK):
            t = tgt_v[pl.ds(j * _LANES, _LANES)]
            rows = base + j * _LANES + lane_iota
            idx_v[j // 8, pl.ds((j % 8) * _LANES, _LANES)] = rows * C + t

        copies = [
            pltpu.async_copy(
                x_hbm.at[idx_v.at[k]],
                got_v.at[pl.ds(k * 128, 128)],
                sem,
            )
            for k in range(NGATHER)
        ]
        for cp in copies:
            cp.wait()

        acc = jnp.zeros((_LANES,), jnp.float32)
        for j in range(NCHUNK):
            acc = acc + got_v[pl.ds(j * _LANES, _LANES)]
        acc_v[...] = acc
        pltpu.sync_copy(acc_v, out_hbm.at[wid])

    return sc_kernel(x_flat, tgt)


def _tc_loss(x, partials, B, C):
    GRID = 32
    BR = B // GRID
    inv_n = 1.0 / (B * C)

    def body(x_ref, p_ref, o_ref):
        i = pl.program_id(0)

        @pl.when(i == 0)
        def _init():
            o_ref[0, 0] = 0.0

        xb = x_ref[...]
        o_ref[0, 0] += jnp.sum(xb * xb)

        @pl.when(i == GRID - 1)
        def _finish():
            gsum = jnp.sum(p_ref[...])
            o_ref[0, 0] = (o_ref[0, 0] - 2.0 * gsum + float(B)) * inv_n

    return pl.pallas_call(
        body,
        grid=(GRID,),
        in_specs=[
            pl.BlockSpec((BR, C), lambda i: (i, 0)),
            pl.BlockSpec(partials.shape, lambda i: (0, 0)),
        ],
        out_specs=pl.BlockSpec(memory_space=pltpu.SMEM),
        out_shape=jax.ShapeDtypeStruct((1, 1), jnp.float32),
    )(x, partials)


def kernel(input_score, target):
    B, C = input_score.shape
    tgt = target.reshape(-1).astype(jnp.int32)
    x_flat = input_score.reshape(B * C)
    partials = _sc_gather_partials(x_flat, tgt, B, C)
    loss = _tc_loss(input_score, partials, B, C)
    return loss[0, 0]


# speedup vs baseline: 1.7862x; 1.5922x over previous
import functools

import jax
import jax.numpy as jnp
from jax import lax
from jax.experimental import pallas as pl
from jax.experimental.pallas import tpu as pltpu
from jax.experimental.pallas import tpu_sc as plsc

_LANES = 16


def _tc_ssq_and_rowvals(x, tgt3, B, C, grid):
    BR = B // grid

    def body(x_ref, t_ref, o_ref, g_ref):
        i = pl.program_id(0)

        @pl.when(i == 0)
        def _init():
            o_ref[0, 0] = 0.0

        xb = x_ref[...]
        o_ref[0, 0] += jnp.sum(xb * xb)

        t = t_ref[0, 0, :]
        cols = lax.broadcasted_iota(jnp.int32, (BR, C), 1)
        picked = jnp.where(cols == t[:, None], xb, 0.0)
        g_ref[0, 0, :] = jnp.sum(picked, axis=1)

    return pl.pallas_call(
        body,
        grid=(grid,),
        in_specs=[
            pl.BlockSpec((BR, C), lambda i: (i, 0)),
            pl.BlockSpec((1, 1, BR), lambda i: (i, 0, 0)),
        ],
        out_specs=[
            pl.BlockSpec(memory_space=pltpu.SMEM),
            pl.BlockSpec((1, 1, BR), lambda i: (i, 0, 0)),
        ],
        out_shape=[
            jax.ShapeDtypeStruct((1, 1), jnp.float32),
            jax.ShapeDtypeStruct((grid, 1, BR), jnp.float32),
        ],
    )(x, tgt3)


def _sc_finalize(gathered, ssq16, B, C):
    NC = 2
    NCHUNK = B // _LANES
    inv_n = 1.0 / (B * C)

    mesh = plsc.VectorSubcoreMesh(core_axis_name="c", subcore_axis_name="s")

    @functools.partial(
        pl.kernel,
        out_type=jax.ShapeDtypeStruct((_LANES,), jnp.float32),
        mesh=mesh,
        scratch_types=[
            pltpu.VMEM((B,), jnp.float32),
            pltpu.VMEM((_LANES,), jnp.float32),
            pltpu.VMEM((_LANES,), jnp.float32),
        ],
    )
    def sc_kernel(g_hbm, s_hbm, out_hbm, g_v, s_v, res_v):
        wid = lax.axis_index("s") * NC + lax.axis_index("c")

        @pl.when(wid == 0)
        def _work():
            pltpu.sync_copy(g_hbm, g_v)
            pltpu.sync_copy(s_hbm, s_v)

            def chunk(j, acc):
                return acc + g_v[pl.ds(j * _LANES, _LANES)]

            acc = lax.fori_loop(0, NCHUNK, chunk,
                                jnp.zeros((_LANES,), jnp.float32))
            lane = lax.iota(jnp.int32, _LANES)
            for sh in (8, 4, 2, 1):
                acc = acc + acc[lax.bitwise_and(lane + sh, _LANES - 1)]
            res_v[...] = (s_v[...] - 2.0 * acc + float(B)) * inv_n
            pltpu.sync_copy(res_v, out_hbm)

    return sc_kernel(gathered, ssq16)


def kernel(input_score, target):
    B, C = input_score.shape
    GRID = 32
    tgt3 = target.reshape(GRID, 1, B // GRID).astype(jnp.int32)
    ssq, gathered3 = _tc_ssq_and_rowvals(input_score, tgt3, B, C, GRID)
    gathered = gathered3.reshape(B)
    ssq16 = jnp.broadcast_to(ssq.reshape(1), (_LANES,))
    out = _sc_finalize(gathered, ssq16, B, C)
    return out[0]


# speedup vs baseline: 1.9347x; 1.0832x over previous
import functools

import jax
import jax.numpy as jnp
from jax import lax
from jax.experimental import pallas as pl
from jax.experimental.pallas import tpu as pltpu
from jax.experimental.pallas import tpu_sc as plsc

_LANES = 16


def _tc_ssq_and_rowvals(x, tgt3, B, C, grid):
    BR = B // grid

    def body(x_ref, t_ref, o_ref, g_ref):
        i = pl.program_id(0)

        @pl.when(i == 0)
        def _init():
            o_ref[0, 0] = 0.0

        xb = x_ref[...]
        o_ref[0, 0] += jnp.sum(xb * xb)

        t = t_ref[0, 0, :]
        cols = lax.broadcasted_iota(jnp.int32, (BR, C), 1)
        picked = jnp.where(cols == t[:, None], xb, 0.0)
        g_ref[0, 0, :] = jnp.sum(picked, axis=1)

    return pl.pallas_call(
        body,
        grid=(grid,),
        in_specs=[
            pl.BlockSpec((BR, C), lambda i: (i, 0)),
            pl.BlockSpec((1, 1, BR), lambda i: (i, 0, 0)),
        ],
        out_specs=[
            pl.BlockSpec(memory_space=pltpu.SMEM),
            pl.BlockSpec((1, 1, BR), lambda i: (i, 0, 0)),
        ],
        out_shape=[
            jax.ShapeDtypeStruct((1, 1), jnp.float32),
            jax.ShapeDtypeStruct((grid, 1, BR), jnp.float32),
        ],
    )(x, tgt3)


def _sc_finalize(gathered, ssq16, B, C):
    NC = 2
    NCHUNK = B // _LANES
    inv_n = 1.0 / (B * C)

    mesh = plsc.VectorSubcoreMesh(core_axis_name="c", subcore_axis_name="s")

    @functools.partial(
        pl.kernel,
        out_type=jax.ShapeDtypeStruct((_LANES,), jnp.float32),
        mesh=mesh,
        scratch_types=[
            pltpu.VMEM((B,), jnp.float32),
            pltpu.VMEM((_LANES,), jnp.float32),
            pltpu.VMEM((_LANES,), jnp.float32),
        ],
    )
    def sc_kernel(g_hbm, s_hbm, out_hbm, g_v, s_v, res_v):
        wid = lax.axis_index("s") * NC + lax.axis_index("c")

        @pl.when(wid == 0)
        def _work():
            pltpu.sync_copy(g_hbm, g_v)
            pltpu.sync_copy(s_hbm, s_v)

            def chunk(j, acc):
                return acc + g_v[pl.ds(j * _LANES, _LANES)]

            acc = lax.fori_loop(0, NCHUNK, chunk,
                                jnp.zeros((_LANES,), jnp.float32))
            lane = lax.iota(jnp.int32, _LANES)
            for sh in (8, 4, 2, 1):
                acc = acc + acc[lax.bitwise_and(lane + sh, _LANES - 1)]
            res_v[...] = (s_v[...] - 2.0 * acc + float(B)) * inv_n
            pltpu.sync_copy(res_v, out_hbm)

    return sc_kernel(gathered, ssq16)


def kernel(input_score, target):
    B, C = input_score.shape
    GRID = 16
    tgt3 = target.reshape(GRID, 1, B // GRID).astype(jnp.int32)
    ssq, gathered3 = _tc_ssq_and_rowvals(input_score, tgt3, B, C, GRID)
    gathered = gathered3.reshape(B)
    ssq16 = jnp.broadcast_to(ssq.reshape(1), (_LANES,))
    out = _sc_finalize(gathered, ssq16, B, C)
    return out[0]


# speedup vs baseline: 1.9991x; 1.0333x over previous
import functools

import jax
import jax.numpy as jnp
from jax import lax
from jax.experimental import pallas as pl
from jax.experimental.pallas import tpu as pltpu
from jax.experimental.pallas import tpu_sc as plsc

_LANES = 16


def _tc_ssq_and_rowvals(x, tgt3, B, C, grid):
    BR = B // grid

    def body(x_ref, t_ref, o_ref, g_ref):
        i = pl.program_id(0)

        @pl.when(i == 0)
        def _init():
            o_ref[0, 0] = 0.0

        xb = x_ref[...]
        o_ref[0, 0] += jnp.sum(xb * xb)

        t = t_ref[0, 0, :]
        cols = lax.broadcasted_iota(jnp.int32, (BR, C), 1)
        picked = jnp.where(cols == t[:, None], xb, 0.0)
        g_ref[0, 0, :] = jnp.sum(picked, axis=1)

    return pl.pallas_call(
        body,
        grid=(grid,),
        in_specs=[
            pl.BlockSpec((BR, C), lambda i: (i, 0)),
            pl.BlockSpec((1, 1, BR), lambda i: (i, 0, 0)),
        ],
        out_specs=[
            pl.BlockSpec(memory_space=pltpu.SMEM),
            pl.BlockSpec((1, 1, BR), lambda i: (i, 0, 0)),
        ],
        out_shape=[
            jax.ShapeDtypeStruct((1, 1), jnp.float32),
            jax.ShapeDtypeStruct((grid, 1, BR), jnp.float32),
        ],
    )(x, tgt3)


def _sc_finalize(gathered, ssq16, B, C):
    NC = 2
    NCHUNK = B // _LANES
    inv_n = 1.0 / (B * C)

    mesh = plsc.VectorSubcoreMesh(core_axis_name="c", subcore_axis_name="s")

    @functools.partial(
        pl.kernel,
        out_type=jax.ShapeDtypeStruct((_LANES,), jnp.float32),
        mesh=mesh,
        scratch_types=[
            pltpu.VMEM((B,), jnp.float32),
            pltpu.VMEM((_LANES,), jnp.float32),
            pltpu.VMEM((_LANES,), jnp.float32),
        ],
    )
    def sc_kernel(g_hbm, s_hbm, out_hbm, g_v, s_v, res_v):
        wid = lax.axis_index("s") * NC + lax.axis_index("c")

        @pl.when(wid == 0)
        def _work():
            pltpu.sync_copy(g_hbm, g_v)
            pltpu.sync_copy(s_hbm, s_v)

            def chunk(j, acc):
                return acc + g_v[pl.ds(j * _LANES, _LANES)]

            acc = lax.fori_loop(0, NCHUNK, chunk,
                                jnp.zeros((_LANES,), jnp.float32))
            lane = lax.iota(jnp.int32, _LANES)
            for sh in (8, 4, 2, 1):
                acc = acc + acc[lax.bitwise_and(lane + sh, _LANES - 1)]
            res_v[...] = (s_v[...] - 2.0 * acc + float(B)) * inv_n
            pltpu.sync_copy(res_v, out_hbm)

    return sc_kernel(gathered, ssq16)


def kernel(input_score, target):
    B, C = input_score.shape
    GRID = 8
    tgt3 = target.reshape(GRID, 1, B // GRID).astype(jnp.int32)
    ssq, gathered3 = _tc_ssq_and_rowvals(input_score, tgt3, B, C, GRID)
    gathered = gathered3.reshape(B)
    ssq16 = jnp.broadcast_to(ssq.reshape(1), (_LANES,))
    out = _sc_finalize(gathered, ssq16, B, C)
    return out[0]
